# baseline (speedup 1.0000x reference)
import functools

import jax
import jax.numpy as jnp
from jax import lax
from jax.experimental import pallas as pl
from jax.experimental.pallas import tpu as pltpu
from jax.experimental.pallas import tpu_sc as plsc

B = 1024
L = 200
D = 128
N = B * L
NC = 2
NS = 16
NW = NC * NS
PER_W = N // NW
CHUNK = 80
NCHUNK = PER_W // CHUNK
LANES = 16
VECS_PER_ROW = D // LANES
NBUF = 5
LEAD = 2

_mesh = plsc.VectorSubcoreMesh(core_axis_name="c", subcore_axis_name="s")


@functools.partial(
    pl.kernel,
    out_type=jax.ShapeDtypeStruct((N, D), jnp.float32),
    mesh=_mesh,
    scratch_types=[
        pltpu.VMEM((PER_W,), jnp.int32),
        pltpu.VMEM((L, D), jnp.float32),
        pltpu.VMEM((CHUNK, D), jnp.float32),
        pltpu.VMEM((CHUNK, D), jnp.float32),
        pltpu.VMEM((CHUNK, D), jnp.float32),
        pltpu.VMEM((CHUNK, D), jnp.float32),
        pltpu.VMEM((CHUNK, D), jnp.float32),
        pltpu.SemaphoreType.DMA((NBUF,)),
        pltpu.SemaphoreType.DMA((NBUF,)),
    ],
)
def _embed(ids_hbm, tok_hbm, pos_hbm, out_hbm,
           idx_v, pos_v, r0, r1, r2, r3, r4, gsem, ssem):
    bufs = [r0, r1, r2, r3, r4]
    wid = lax.axis_index("s") * NC + lax.axis_index("c")
    base = wid * PER_W

    pltpu.sync_copy(ids_hbm.at[pl.ds(base, PER_W)], idx_v)
    pltpu.sync_copy(pos_hbm.at[pl.ds(0, L)], pos_v)

    def issue_gather(j, b):
        idx_slice = idx_v.at[pl.ds(j * CHUNK, CHUNK)]
        pltpu.async_copy(tok_hbm.at[idx_slice], bufs[b], gsem.at[b])

    def wait_gather(b):
        idx_slice = idx_v.at[pl.ds(0, CHUNK)]
        pltpu.make_async_copy(tok_hbm.at[idx_slice], bufs[b], gsem.at[b]).wait()

    def issue_scatter(j, b):
        out_slice = out_hbm.at[pl.ds(base + j * CHUNK, CHUNK)]
        pltpu.async_copy(bufs[b], out_slice, ssem.at[b])

    def wait_scatter(b):
        out_slice = out_hbm.at[pl.ds(0, CHUNK)]
        pltpu.make_async_copy(bufs[b], out_slice, ssem.at[b]).wait()

    for b in range(LEAD):
        issue_gather(b, b)

    def outer_body(jj, carry):
        j0 = jj * NBUF
        for b in range(NBUF):
            j = j0 + b
            b2 = (b + LEAD) % NBUF

            @pl.when(j + LEAD < NCHUNK)
            def _():
                @pl.when(j + LEAD >= NBUF)
                def _():
                    wait_scatter(b2)
                issue_gather(j + LEAD, b2)

            wait_gather(b)

            prow0 = lax.rem(j * CHUNK, L)
            rows = bufs[b]

            @plsc.parallel_loop(0, CHUNK, step=1, unroll=8)
            def _(r):
                p = prow0 + r
                p = jnp.where(p >= L, p - L, p)
                for c in range(VECS_PER_ROW):
                    sl = pl.ds(c * LANES, LANES)
                    plsc.addupdate(rows.at[r, sl], pos_v[p, sl])

            issue_scatter(j, b)
        return carry

    lax.fori_loop(0, NCHUNK // NBUF, outer_body, 0)

    for b in range(NBUF):
        wait_scatter(b)


def kernel(input_ids, token_table, pos_table):
    ids_flat = input_ids.reshape(-1).astype(jnp.int32)
    out = _embed(ids_flat, token_table, pos_table)
    return out.reshape(B, L, D)


# speedup vs baseline: 1.0151x; 1.0151x over previous
import functools

import jax
import jax.numpy as jnp
from jax import lax
from jax.experimental import pallas as pl
from jax.experimental.pallas import tpu as pltpu
from jax.experimental.pallas import tpu_sc as plsc

B = 1024
L = 200
D = 128
N = B * L
NC = 2
NS = 16
NW = NC * NS
PER_W = N // NW
CHUNK = 80
NCHUNK = PER_W // CHUNK
LANES = 16
VECS_PER_ROW = D // LANES
NBUF = 5
LEAD = 2

_mesh = plsc.VectorSubcoreMesh(core_axis_name="c", subcore_axis_name="s")


@functools.partial(
    pl.kernel,
    out_type=jax.ShapeDtypeStruct((N, D), jnp.float32),
    mesh=_mesh,
    scratch_types=[
        pltpu.VMEM((PER_W,), jnp.int32),
        pltpu.VMEM((L, D), jnp.float32),
        pltpu.VMEM((CHUNK, D), jnp.float32),
        pltpu.VMEM((CHUNK, D), jnp.float32),
        pltpu.VMEM((CHUNK, D), jnp.float32),
        pltpu.VMEM((CHUNK, D), jnp.float32),
        pltpu.VMEM((CHUNK, D), jnp.float32),
        pltpu.SemaphoreType.DMA((NBUF,)),
        pltpu.SemaphoreType.DMA((NBUF,)),
        pltpu.SemaphoreType.DMA,
    ],
)
def _embed(ids_hbm, tok_hbm, pos_hbm, out_hbm,
           idx_v, pos_v, r0, r1, r2, r3, r4, gsem, ssem, psem):
    bufs = [r0, r1, r2, r3, r4]
    wid = lax.axis_index("s") * NC + lax.axis_index("c")
    base = wid * PER_W

    pltpu.sync_copy(ids_hbm.at[pl.ds(base, PER_W)], idx_v)
    pos_copy = pltpu.async_copy(pos_hbm.at[pl.ds(0, L)], pos_v, psem)

    def issue_gather(j, b):
        idx_slice = idx_v.at[pl.ds(j * CHUNK, CHUNK)]
        pltpu.async_copy(tok_hbm.at[idx_slice], bufs[b], gsem.at[b])

    def wait_gather(b):
        idx_slice = idx_v.at[pl.ds(0, CHUNK)]
        pltpu.make_async_copy(tok_hbm.at[idx_slice], bufs[b], gsem.at[b]).wait()

    def issue_scatter(j, b):
        out_slice = out_hbm.at[pl.ds(base + j * CHUNK, CHUNK)]
        pltpu.async_copy(bufs[b], out_slice, ssem.at[b])

    def wait_scatter(b):
        out_slice = out_hbm.at[pl.ds(0, CHUNK)]
        pltpu.make_async_copy(bufs[b], out_slice, ssem.at[b]).wait()

    for b in range(LEAD):
        issue_gather(b, b)
    pos_copy.wait()

    def outer_body(jj, carry):
        j0 = jj * NBUF
        for b in range(NBUF):
            j = j0 + b
            b2 = (b + LEAD) % NBUF

            @pl.when(j + LEAD < NCHUNK)
            def _():
                @pl.when(j + LEAD >= NBUF)
                def _():
                    wait_scatter(b2)
                issue_gather(j + LEAD, b2)

            wait_gather(b)

            prow0 = lax.rem(j * CHUNK, L)
            rows = bufs[b]

            @plsc.parallel_loop(0, CHUNK, step=1, unroll=4)
            def _(r):
                p = prow0 + r
                p = jnp.where(p >= L, p - L, p)
                for c in range(VECS_PER_ROW):
                    sl = pl.ds(c * LANES, LANES)
                    plsc.addupdate(rows.at[r, sl], pos_v[p, sl])

            issue_scatter(j, b)
        return carry

    lax.fori_loop(0, NCHUNK // NBUF, outer_body, 0)

    for b in range(NBUF):
        wait_scatter(b)


def kernel(input_ids, token_table, pos_table):
    ids_flat = input_ids.reshape(-1).astype(jnp.int32)
    out = _embed(ids_flat, token_table, pos_table)
    return out.reshape(B, L, D)
